# Initial kernel scaffold; baseline (speedup 1.0000x reference)
import jax
import jax.numpy as jnp
from jax.experimental import pallas as pl


def kernel(x, task_labels, muy, log_alpha, epsilon):
    raise NotImplementedError("write your pallas kernel here")


# speedup vs baseline: 1.2037x; 1.2037x over previous
import functools

import jax
import jax.numpy as jnp
from jax import lax
from jax.experimental import pallas as pl
from jax.experimental.pallas import tpu as pltpu
from jax.experimental.pallas import tpu_sc as plsc

B = 16384
D = 128
LANES = 16
NW = 32
ROWS_PER_W = B // NW
CHUNK = 64
NCHUNK = ROWS_PER_W // CHUNK


def _body(x_hbm, lab_hbm, tab_hbm, eps_hbm, out_hbm,
          idx_v, rows_v, x_v, eps_v, sem_t, sem_x, sem_e):
    cid = lax.axis_index("c")
    sid = lax.axis_index("s")
    wid = sid * 2 + cid
    wbase = wid * ROWS_PER_W

    for ch in range(NCHUNK):
        base = wbase + ch * CHUNK
        pltpu.sync_copy(lab_hbm.at[pl.ds(base, CHUNK)], idx_v)
        cp_t = pltpu.async_copy(tab_hbm.at[idx_v], rows_v, sem_t)
        cp_x = pltpu.async_copy(x_hbm.at[pl.ds(base, CHUNK)], x_v, sem_x)
        cp_e = pltpu.async_copy(eps_hbm.at[pl.ds(base, CHUNK)], eps_v, sem_e)
        cp_t.wait()
        cp_x.wait()
        cp_e.wait()

        def row_body(r, carry):
            for j in range(D // LANES):
                mu = rows_v[r, pl.ds(j * LANES, LANES)]
                la = rows_v[r, pl.ds(D + j * LANES, LANES)]
                xv = x_v[r, pl.ds(j * LANES, LANES)]
                ev = eps_v[r, pl.ds(j * LANES, LANES)]
                x_v[r, pl.ds(j * LANES, LANES)] = xv * (ev * jnp.exp(la) + mu)
            return carry

        lax.fori_loop(0, CHUNK, row_body, 0)
        pltpu.sync_copy(x_v, out_hbm.at[pl.ds(base, CHUNK)])


@jax.jit
def _gauss_dropout_sc(x, labels, tab, epsilon):
    mesh = plsc.VectorSubcoreMesh(core_axis_name="c", subcore_axis_name="s")
    kfn = functools.partial(
        pl.kernel,
        mesh=mesh,
        out_type=jax.ShapeDtypeStruct((B, D), jnp.float32),
        scratch_types=[
            pltpu.VMEM((CHUNK,), jnp.int32),
            pltpu.VMEM((CHUNK, 2 * D), jnp.float32),
            pltpu.VMEM((CHUNK, D), jnp.float32),
            pltpu.VMEM((CHUNK, D), jnp.float32),
            pltpu.SemaphoreType.DMA,
            pltpu.SemaphoreType.DMA,
            pltpu.SemaphoreType.DMA,
        ],
    )(_body)
    return kfn(x, labels, tab, epsilon)


def kernel(x, task_labels, muy, log_alpha, epsilon):
    tab = jnp.concatenate([muy, log_alpha], axis=1)
    labels = task_labels.astype(jnp.int32)
    return _gauss_dropout_sc(x, labels, tab, epsilon)


# speedup vs baseline: 2.4809x; 2.0611x over previous
import functools

import jax
import jax.numpy as jnp
from jax import lax
from jax.experimental import pallas as pl
from jax.experimental.pallas import tpu as pltpu
from jax.experimental.pallas import tpu_sc as plsc

B = 16384
D = 128
LANES = 16
NW = 32
NSUB = 16
ROWS_PER_W = B // NW
CHUNK = 64
NCHUNK = ROWS_PER_W // CHUNK
TPAD = 1024
TROWS = TPAD // NSUB


def _body(x_hbm, lab_hbm, mu_hbm, la_hbm, eps_hbm, out_hbm,
          idx_v, mu_v, al_v, x_v, eps_v, mu_tab, al_tab,
          sem_g1, sem_g2, sem_x, sem_e):
    cid = lax.axis_index("c")
    sid = lax.axis_index("s")
    wid = sid * 2 + cid
    wbase = wid * ROWS_PER_W

    r0 = sid * TROWS
    pltpu.sync_copy(mu_hbm.at[pl.ds(r0, TROWS)], mu_tab.at[pl.ds(r0, TROWS)])
    pltpu.sync_copy(la_hbm.at[pl.ds(r0, TROWS)], eps_v)

    def exp_row(r, carry):
        for j in range(D // LANES):
            sl = pl.ds(j * LANES, LANES)
            eps_v[r, sl] = jnp.exp(eps_v[r, sl])
        return carry

    lax.fori_loop(0, TROWS, exp_row, 0)
    pltpu.sync_copy(eps_v, al_tab.at[pl.ds(r0, TROWS)])
    plsc.subcore_barrier()

    for ch in range(NCHUNK):
        base = wbase + ch * CHUNK
        pltpu.sync_copy(lab_hbm.at[pl.ds(base, CHUNK)], idx_v)
        cp_m = pltpu.async_copy(mu_tab.at[idx_v], mu_v, sem_g1)
        cp_a = pltpu.async_copy(al_tab.at[idx_v], al_v, sem_g2)
        cp_x = pltpu.async_copy(x_hbm.at[pl.ds(base, CHUNK)], x_v, sem_x)
        cp_e = pltpu.async_copy(eps_hbm.at[pl.ds(base, CHUNK)], eps_v, sem_e)
        cp_m.wait()
        cp_a.wait()
        cp_x.wait()
        cp_e.wait()

        def row_body(r, carry):
            for j in range(D // LANES):
                sl = pl.ds(j * LANES, LANES)
                x_v[r, sl] = x_v[r, sl] * (eps_v[r, sl] * al_v[r, sl] + mu_v[r, sl])
            return carry

        lax.fori_loop(0, CHUNK, row_body, 0)
        pltpu.sync_copy(x_v, out_hbm.at[pl.ds(base, CHUNK)])


@jax.jit
def _gauss_dropout_sc(x, labels, mu_p, la_p, epsilon):
    mesh = plsc.VectorSubcoreMesh(core_axis_name="c", subcore_axis_name="s")
    kfn = functools.partial(
        pl.kernel,
        mesh=mesh,
        out_type=jax.ShapeDtypeStruct((B, D), jnp.float32),
        scratch_types=[
            pltpu.VMEM((CHUNK,), jnp.int32),
            pltpu.VMEM((CHUNK, D), jnp.float32),
            pltpu.VMEM((CHUNK, D), jnp.float32),
            pltpu.VMEM((CHUNK, D), jnp.float32),
            pltpu.VMEM((CHUNK, D), jnp.float32),
            pltpu.VMEM_SHARED((TPAD, D), jnp.float32),
            pltpu.VMEM_SHARED((TPAD, D), jnp.float32),
            pltpu.SemaphoreType.DMA,
            pltpu.SemaphoreType.DMA,
            pltpu.SemaphoreType.DMA,
            pltpu.SemaphoreType.DMA,
        ],
    )(_body)
    return kfn(x, labels, mu_p, la_p, epsilon)


def kernel(x, task_labels, muy, log_alpha, epsilon):
    labels = task_labels.astype(jnp.int32)
    pad = ((0, TPAD - muy.shape[0]), (0, 0))
    mu_p = jnp.pad(muy, pad)
    la_p = jnp.pad(log_alpha, pad)
    return _gauss_dropout_sc(x, labels, mu_p, la_p, epsilon)


# speedup vs baseline: 3.1192x; 1.2573x over previous
import functools

import jax
import jax.numpy as jnp
from jax import lax
from jax.experimental import pallas as pl
from jax.experimental.pallas import tpu as pltpu
from jax.experimental.pallas import tpu_sc as plsc

B = 16384
D = 128
LANES = 16
NW = 32
NSUB = 16
ROWS_PER_W = B // NW
CHUNK = 64
NCHUNK = ROWS_PER_W // CHUNK
TPAD = 1024
TROWS = TPAD // NSUB


def _body(x_hbm, lab_hbm, mu_hbm, la_hbm, eps_hbm, out_hbm,
          idx0, idx1, mu0, mu1, al0, al1, x0, x1, e0, e1, o0, o1,
          stage_v, mu_tab, al_tab,
          sm0, sm1, sa0, sa1, sx0, sx1, se0, se1, sem_out0, sem_out1):
    idx_v = (idx0, idx1)
    mu_v = (mu0, mu1)
    al_v = (al0, al1)
    x_v = (x0, x1)
    eps_v = (e0, e1)
    out_v = (o0, o1)
    sem_m = (sm0, sm1)
    sem_a = (sa0, sa1)
    sem_x = (sx0, sx1)
    sem_e = (se0, se1)
    sem_out = (sem_out0, sem_out1)

    cid = lax.axis_index("c")
    sid = lax.axis_index("s")
    wid = sid * 2 + cid
    wbase = wid * ROWS_PER_W

    r0 = sid * TROWS
    pltpu.sync_copy(mu_hbm.at[pl.ds(r0, TROWS)], mu_tab.at[pl.ds(r0, TROWS)])
    pltpu.sync_copy(la_hbm.at[pl.ds(r0, TROWS)], stage_v)

    def exp_row(r, carry):
        for j in range(D // LANES):
            sl = pl.ds(j * LANES, LANES)
            stage_v[r, sl] = jnp.exp(stage_v[r, sl])
        return carry

    lax.fori_loop(0, TROWS, exp_row, 0)
    pltpu.sync_copy(stage_v, al_tab.at[pl.ds(r0, TROWS)])
    plsc.subcore_barrier()

    copies = [None, None]
    outcp = [None, None]

    def start(ch):
        p = ch % 2
        base = wbase + ch * CHUNK
        pltpu.sync_copy(lab_hbm.at[pl.ds(base, CHUNK)], idx_v[p])
        copies[p] = (
            pltpu.async_copy(mu_tab.at[idx_v[p]], mu_v[p], sem_m[p]),
            pltpu.async_copy(al_tab.at[idx_v[p]], al_v[p], sem_a[p]),
            pltpu.async_copy(x_hbm.at[pl.ds(base, CHUNK)], x_v[p], sem_x[p]),
            pltpu.async_copy(eps_hbm.at[pl.ds(base, CHUNK)], eps_v[p], sem_e[p]),
        )

    start(0)
    for ch in range(NCHUNK):
        p = ch % 2
        if ch + 1 < NCHUNK:
            start(ch + 1)
        for c in copies[p]:
            c.wait()
        if outcp[p] is not None:
            outcp[p].wait()

        xv, ev, av, mv, ov = x_v[p], eps_v[p], al_v[p], mu_v[p], out_v[p]

        def row_body(r, carry):
            for j in range(D // LANES):
                sl = pl.ds(j * LANES, LANES)
                ov[r, sl] = xv[r, sl] * (ev[r, sl] * av[r, sl] + mv[r, sl])
            return carry

        lax.fori_loop(0, CHUNK, row_body, 0)
        outcp[p] = pltpu.async_copy(
            out_v[p], out_hbm.at[pl.ds(wbase + ch * CHUNK, CHUNK)], sem_out[p])
    for p in range(2):
        if outcp[p] is not None:
            outcp[p].wait()


@jax.jit
def _gauss_dropout_sc(x, labels, mu_p, la_p, epsilon):
    mesh = plsc.VectorSubcoreMesh(core_axis_name="c", subcore_axis_name="s")
    buf = lambda: pltpu.VMEM((CHUNK, D), jnp.float32)
    kfn = functools.partial(
        pl.kernel,
        mesh=mesh,
        out_type=jax.ShapeDtypeStruct((B, D), jnp.float32),
        scratch_types=[
            pltpu.VMEM((CHUNK,), jnp.int32), pltpu.VMEM((CHUNK,), jnp.int32),
            buf(), buf(), buf(), buf(), buf(), buf(), buf(), buf(), buf(), buf(),
            pltpu.VMEM((TROWS, D), jnp.float32),
            pltpu.VMEM_SHARED((TPAD, D), jnp.float32),
            pltpu.VMEM_SHARED((TPAD, D), jnp.float32),
            pltpu.SemaphoreType.DMA, pltpu.SemaphoreType.DMA,
            pltpu.SemaphoreType.DMA, pltpu.SemaphoreType.DMA,
            pltpu.SemaphoreType.DMA, pltpu.SemaphoreType.DMA,
            pltpu.SemaphoreType.DMA, pltpu.SemaphoreType.DMA,
            pltpu.SemaphoreType.DMA, pltpu.SemaphoreType.DMA,
        ],
    )(_body)
    return kfn(x, labels, mu_p, la_p, epsilon)


def kernel(x, task_labels, muy, log_alpha, epsilon):
    labels = task_labels.astype(jnp.int32)
    pad = ((0, TPAD - muy.shape[0]), (0, 0))
    mu_p = jnp.pad(muy, pad)
    la_p = jnp.pad(log_alpha, pad)
    return _gauss_dropout_sc(x, labels, mu_p, la_p, epsilon)
